# baseline (speedup 1.0000x reference)
import functools

import jax
import jax.numpy as jnp
from jax import lax
from jax.experimental import pallas as pl
from jax.experimental.pallas import tpu as pltpu
from jax.experimental.pallas import tpu_sc as plsc

T, H, W, C, A = 128, 64, 64, 128, 64
B = T * A
NC, NS, L = 2, 16, 16
NW = NC * NS
APW = A // NW
BPW = B // NW
NQ = 4
QR = BPW // NQ
TPA = T // (NQ // APW)


def _make_sc_gather():
    mesh = plsc.VectorSubcoreMesh(core_axis_name="c", subcore_axis_name="s")

    @functools.partial(
        pl.kernel,
        out_type=jax.ShapeDtypeStruct((B, C), jnp.float32),
        mesh=mesh,
        scratch_types=[
            pltpu.VMEM((APW, 2, T), jnp.int32),
            pltpu.VMEM((NQ, QR), jnp.int32),
            pltpu.VMEM((NQ, QR), jnp.int32),
            pltpu.VMEM((BPW, C), jnp.float32),
            pltpu.SemaphoreType.DMA,
            pltpu.SemaphoreType.DMA,
            pltpu.SemaphoreType.DMA,
            pltpu.SemaphoreType.DMA,
            pltpu.SemaphoreType.DMA,
            pltpu.SemaphoreType.DMA,
        ],
    )
    def gather_kernel(table, rc, out, rc_v, idx_v, oidx_v, feat_v,
                      csem, g0, g1, g2, g3, wsem):
        gsems = (g0, g1, g2, g3)
        wid = lax.axis_index("s") * NC + lax.axis_index("c")

        cps = [pltpu.async_copy(rc.at[pl.ds(APW * wid + a, 1)],
                                rc_v.at[pl.ds(a, 1)],
                                csem if a == 0 else wsem)
               for a in range(APW)]

        lane = lax.iota(jnp.int32, L)
        one = jnp.full((L,), 1.0, jnp.float32)
        zero = jnp.full((L,), 0.0, jnp.float32)
        copies = []
        for q in range(NQ):
            al = q // (NQ // APW)
            t0 = (q % (NQ // APW)) * TPA
            if q % (NQ // APW) == 0:
                cps[al].wait()
            for kt in range(QR // L):
                tv = t0 + kt * L + lane
                rraw = rc_v[al, 0, pl.ds(t0 + kt * L, L)]
                craw = rc_v[al, 1, pl.ds(t0 + kt * L, L)]
                flat = tv * (H * W) + (rraw & 63) * W + (craw & 63)
                idx_v[q, pl.ds(kt * L, L)] = flat
                oidx_v[q, pl.ds(kt * L, L)] = tv * A + (APW * wid + al)
            copies.append(pltpu.async_copy(
                table.at[idx_v.at[q]], feat_v.at[pl.ds(q * QR, QR)],
                gsems[q]))

        wcopies = []
        for q in range(NQ):
            copies[q].wait()

            def mul_group(g16, _, q=q):
                al = q // (NQ // APW)
                t0 = (q % (NQ // APW)) * TPA
                raw = rc_v[al, 0, pl.ds(t0 + g16 * L, L)]
                mv = jnp.where(raw >= 0, one, zero)
                for j in range(L):
                    mrow = jnp.broadcast_to(
                        lax.slice(mv, (j,), (j + 1,)), (L,))
                    row = q * QR + g16 * L + j
                    for cc in range(C // L):
                        feat_v[row, pl.ds(cc * L, L)] = (
                            feat_v[row, pl.ds(cc * L, L)] * mrow)
                return 0

            lax.fori_loop(0, QR // L, mul_group, 0)
            wcopies.append(pltpu.async_copy(
                feat_v.at[pl.ds(q * QR, QR)], out.at[oidx_v.at[q]], wsem))

        for wc in wcopies:
            wc.wait()

    return gather_kernel


_sc_gather = _make_sc_gather()


def kernel(feature_map, agent_positions, mask):
    table = feature_map.reshape(T * H * W, C)
    signbit = jnp.int32(-2147483648)
    rc = jnp.transpose(
        jnp.where(mask[:, :, None], agent_positions,
                  agent_positions | signbit),
        (1, 2, 0))
    out = _sc_gather(table, rc)
    return out.reshape(T, A, C)


# speedup vs baseline: 1.0167x; 1.0167x over previous
import functools

import jax
import jax.numpy as jnp
from jax import lax
from jax.experimental import pallas as pl
from jax.experimental.pallas import tpu as pltpu
from jax.experimental.pallas import tpu_sc as plsc

T, H, W, C, A = 128, 64, 64, 128, 64
B = T * A
NC, NS, L = 2, 16, 16
NW = NC * NS
APW = A // NW
BPW = B // NW
NQ = 2
QR = BPW // NQ
TPA = T // (NQ // APW)


def _make_sc_gather():
    mesh = plsc.VectorSubcoreMesh(core_axis_name="c", subcore_axis_name="s")

    @functools.partial(
        pl.kernel,
        out_type=jax.ShapeDtypeStruct((B, C), jnp.float32),
        mesh=mesh,
        scratch_types=[
            pltpu.VMEM((APW, 2, T), jnp.int32),
            pltpu.VMEM((NQ, QR), jnp.int32),
            pltpu.VMEM((NQ, QR), jnp.int32),
            pltpu.VMEM((BPW, C), jnp.float32),
            pltpu.SemaphoreType.DMA,
            pltpu.SemaphoreType.DMA,
            pltpu.SemaphoreType.DMA,
            pltpu.SemaphoreType.DMA,
        ],
    )
    def gather_kernel(table, rc, out, rc_v, idx_v, oidx_v, feat_v,
                      csem, g0, g1, wsem):
        gsems = (g0, g1)
        wid = lax.axis_index("s") * NC + lax.axis_index("c")

        cps = [pltpu.async_copy(rc.at[pl.ds(APW * wid + a, 1)],
                                rc_v.at[pl.ds(a, 1)],
                                csem if a == 0 else wsem)
               for a in range(APW)]

        lane = lax.iota(jnp.int32, L)
        one = jnp.full((L,), 1.0, jnp.float32)
        zero = jnp.full((L,), 0.0, jnp.float32)
        copies = []
        for q in range(NQ):
            al = q // (NQ // APW)
            t0 = (q % (NQ // APW)) * TPA
            if q % (NQ // APW) == 0:
                cps[al].wait()
            for kt in range(QR // L):
                tv = t0 + kt * L + lane
                rraw = rc_v[al, 0, pl.ds(t0 + kt * L, L)]
                craw = rc_v[al, 1, pl.ds(t0 + kt * L, L)]
                flat = tv * (H * W) + (rraw & 63) * W + (craw & 63)
                idx_v[q, pl.ds(kt * L, L)] = flat
                oidx_v[q, pl.ds(kt * L, L)] = tv * A + (APW * wid + al)
            copies.append(pltpu.async_copy(
                table.at[idx_v.at[q]], feat_v.at[pl.ds(q * QR, QR)],
                gsems[q]))

        wcopies = []
        for q in range(NQ):
            copies[q].wait()

            def mul_group(g16, _, q=q):
                al = q // (NQ // APW)
                t0 = (q % (NQ // APW)) * TPA
                raw = rc_v[al, 0, pl.ds(t0 + g16 * L, L)]
                mv = jnp.where(raw >= 0, one, zero)
                for j in range(L):
                    mrow = jnp.broadcast_to(
                        lax.slice(mv, (j,), (j + 1,)), (L,))
                    row = q * QR + g16 * L + j
                    for cc in range(C // L):
                        feat_v[row, pl.ds(cc * L, L)] = (
                            feat_v[row, pl.ds(cc * L, L)] * mrow)
                return 0

            lax.fori_loop(0, QR // L, mul_group, 0)
            wcopies.append(pltpu.async_copy(
                feat_v.at[pl.ds(q * QR, QR)], out.at[oidx_v.at[q]], wsem))

        for wc in wcopies:
            wc.wait()

    return gather_kernel


_sc_gather = _make_sc_gather()


def kernel(feature_map, agent_positions, mask):
    table = feature_map.reshape(T * H * W, C)
    signbit = jnp.int32(-2147483648)
    rc = jnp.transpose(
        jnp.where(mask[:, :, None], agent_positions,
                  agent_positions | signbit),
        (1, 2, 0))
    out = _sc_gather(table, rc)
    return out.reshape(T, A, C)
